# baseline (speedup 1.0000x reference)
import jax
import jax.numpy as jnp
from jax.experimental import pallas as pl
from jax.experimental.pallas import tpu as pltpu

_BATCH = 16
_N_NODES = 512
_HIDDEN = 768
_SPLIT = 3
_HS = _HIDDEN // _SPLIT


def _tc_body(node_ref, tok_ref, out_ref):
    out_ref[0, 0:1, :] = tok_ref[...]
    out_ref[0, 1:_N_NODES + 1, :] = node_ref[0]


@jax.jit
def _tc_call(node_feature, graph_token):
    return pl.pallas_call(
        _tc_body,
        grid=(_BATCH, _SPLIT),
        in_specs=[
            pl.BlockSpec((1, _N_NODES, _HS), lambda b, j: (b, 0, j)),
            pl.BlockSpec((1, _HS), lambda b, j: (0, j)),
        ],
        out_specs=pl.BlockSpec((1, _N_NODES + 1, _HS),
                               lambda b, j: (b, 0, j)),
        out_shape=jax.ShapeDtypeStruct((_BATCH, _N_NODES + 1, _HIDDEN),
                                       jnp.float32),
        compiler_params=pltpu.CompilerParams(
            dimension_semantics=("parallel", "parallel"),
        ),
    )(node_feature, graph_token)


def kernel(node_feature, graph_token):
    return _tc_call(node_feature, graph_token)


# speedup vs baseline: 1.4772x; 1.4772x over previous
import jax
import jax.numpy as jnp
from jax.experimental import pallas as pl
from jax.experimental.pallas import tpu as pltpu

_BATCH = 16
_N_NODES = 512
_HIDDEN = 768
_BB = 2


def _tc_body(node_ref, tok_ref, out_ref):
    for i in range(_BB):
        out_ref[i, 0:1, :] = tok_ref[...]
        out_ref[i, 1:_N_NODES + 1, :] = node_ref[i]


@jax.jit
def _tc_call(node_feature, graph_token):
    return pl.pallas_call(
        _tc_body,
        grid=(_BATCH // _BB,),
        in_specs=[
            pl.BlockSpec((_BB, _N_NODES, _HIDDEN), lambda b: (b, 0, 0)),
            pl.BlockSpec((1, _HIDDEN), lambda b: (0, 0)),
        ],
        out_specs=pl.BlockSpec((_BB, _N_NODES + 1, _HIDDEN),
                               lambda b: (b, 0, 0)),
        out_shape=jax.ShapeDtypeStruct((_BATCH, _N_NODES + 1, _HIDDEN),
                                       jnp.float32),
        compiler_params=pltpu.CompilerParams(
            dimension_semantics=("parallel",),
        ),
    )(node_feature, graph_token)


def kernel(node_feature, graph_token):
    return _tc_call(node_feature, graph_token)


# speedup vs baseline: 1.5243x; 1.0319x over previous
import jax
import jax.numpy as jnp
from jax.experimental import pallas as pl
from jax.experimental.pallas import tpu as pltpu

_BATCH = 16
_N_NODES = 512
_HIDDEN = 768
_BB = 4


def _tc_body(node_ref, tok_ref, out_ref):
    for i in range(_BB):
        out_ref[i, 0:1, :] = tok_ref[...]
        out_ref[i, 1:_N_NODES + 1, :] = node_ref[i]


@jax.jit
def _tc_call(node_feature, graph_token):
    return pl.pallas_call(
        _tc_body,
        grid=(_BATCH // _BB,),
        in_specs=[
            pl.BlockSpec((_BB, _N_NODES, _HIDDEN), lambda b: (b, 0, 0)),
            pl.BlockSpec((1, _HIDDEN), lambda b: (0, 0)),
        ],
        out_specs=pl.BlockSpec((_BB, _N_NODES + 1, _HIDDEN),
                               lambda b: (b, 0, 0)),
        out_shape=jax.ShapeDtypeStruct((_BATCH, _N_NODES + 1, _HIDDEN),
                                       jnp.float32),
        compiler_params=pltpu.CompilerParams(
            dimension_semantics=("parallel",),
        ),
    )(node_feature, graph_token)


def kernel(node_feature, graph_token):
    return _tc_call(node_feature, graph_token)


# speedup vs baseline: 1.5693x; 1.0295x over previous
import jax
import jax.numpy as jnp
from jax.experimental import pallas as pl
from jax.experimental.pallas import tpu as pltpu

_BATCH = 16
_N_NODES = 512
_HIDDEN = 768
_BB = 8


def _tc_body(node_ref, tok_ref, out_ref):
    for i in range(_BB):
        out_ref[i, 0:1, :] = tok_ref[...]
        out_ref[i, 1:_N_NODES + 1, :] = node_ref[i]


@jax.jit
def _tc_call(node_feature, graph_token):
    return pl.pallas_call(
        _tc_body,
        grid=(_BATCH // _BB,),
        in_specs=[
            pl.BlockSpec((_BB, _N_NODES, _HIDDEN), lambda b: (b, 0, 0)),
            pl.BlockSpec((1, _HIDDEN), lambda b: (0, 0)),
        ],
        out_specs=pl.BlockSpec((_BB, _N_NODES + 1, _HIDDEN),
                               lambda b: (b, 0, 0)),
        out_shape=jax.ShapeDtypeStruct((_BATCH, _N_NODES + 1, _HIDDEN),
                                       jnp.float32),
        compiler_params=pltpu.CompilerParams(
            dimension_semantics=("parallel",),
        ),
    )(node_feature, graph_token)


def kernel(node_feature, graph_token):
    return _tc_call(node_feature, graph_token)
